# baseline (speedup 1.0000x reference)
import functools

import jax
import jax.numpy as jnp
from jax import lax
from jax.experimental import pallas as pl
from jax.experimental.pallas import tpu as pltpu
from jax.experimental.pallas import tpu_sc as plsc

N, K, V = 16384, 64, 1000
BN = 512

_TINY = 1.1754943508222875e-38


def _gumbel_w_block(i):
    rowoff = (jax.lax.broadcasted_iota(jnp.uint32, (BN, 1), 0)
              + jnp.uint32(i * BN)) * jnp.uint32(V)
    col = jax.lax.broadcasted_iota(jnp.uint32, (BN, V), 1)
    c2 = rowoff + col
    k1 = jnp.uint32(0)
    k2 = jnp.uint32(42)
    ks0, ks1, ks2 = k1, k2, k1 ^ k2 ^ jnp.uint32(0x1BD11BDA)
    x0 = jnp.full((BN, V), ks0, jnp.uint32)
    x1 = c2 + ks1

    def rnd(x0, x1, r):
        x0 = x0 + x1
        x1 = (x1 << jnp.uint32(r)) | (x1 >> jnp.uint32(32 - r))
        return x0, x0 ^ x1

    rot_a = (13, 15, 26, 6)
    rot_b = (17, 29, 16, 24)
    for r in rot_a:
        x0, x1 = rnd(x0, x1, r)
    x0, x1 = x0 + ks1, x1 + ks2 + jnp.uint32(1)
    for r in rot_b:
        x0, x1 = rnd(x0, x1, r)
    x0, x1 = x0 + ks2, x1 + ks0 + jnp.uint32(2)
    for r in rot_a:
        x0, x1 = rnd(x0, x1, r)
    x0, x1 = x0 + ks0, x1 + ks1 + jnp.uint32(3)
    for r in rot_b:
        x0, x1 = rnd(x0, x1, r)
    x0, x1 = x0 + ks1, x1 + ks2 + jnp.uint32(4)
    for r in rot_a:
        x0, x1 = rnd(x0, x1, r)
    x0, x1 = x0 + ks2, x1 + ks0 + jnp.uint32(5)

    bits = x0 ^ x1
    fb = (bits >> jnp.uint32(9)) | jnp.uint32(0x3F800000)
    u01 = jax.lax.bitcast_convert_type(fb, jnp.float32) - jnp.float32(1.0)
    u = u01 + jnp.float32(_TINY)
    return -jnp.log(u)


def _tc_body(a_ref, prod_ref, z_ref, kap_ref, g_ref, rho_ref, noi_ref,
             m_ref, gamma_ref, p_ref, A_ref, xn_ref):
    ea = jnp.exp(a_ref[...])
    mm = lax.dot_general(ea, prod_ref[...], (((1,), (1,)), ((), ())),
                         preferred_element_type=jnp.float32)
    A_ref[...] = jnp.log(mm + 1.0)

    u = z_ref[...] + kap_ref[...] * g_ref[...] + gamma_ref[0, 0] * rho_ref[...] + noi_ref[...]

    m = (m_ref[...] != 0)[:, :V]
    xmask = jnp.where(m, u, -jnp.inf)
    xmax = jnp.max(xmask, axis=1, keepdims=True)
    e = jnp.exp(xmask - xmax)
    s = jnp.sum(e, axis=1, keepdims=True)
    p = e / s
    p_ref[...] = p

    w = _gumbel_w_block(pl.program_id(0))
    r = jnp.where(m, (p + 1e-20) / w, 0.0)
    rmax = jnp.max(r, axis=1, keepdims=True)
    ii = lax.broadcasted_iota(jnp.int32, (BN, V), 1)
    idx = jnp.min(jnp.where(r == rmax, ii, jnp.int32(2**30)), axis=1)
    xn_ref[...] = idx[:, None]


VP = 1024
VW = VP // 4
_NW = 32
_RPW = N // _NW
_CH = 128


def _sc_gather(x_it_hbm, adj_hbm, out_hbm, idx_v, rows_v, sem):
    wid = lax.axis_index("s") * 2 + lax.axis_index("c")
    base = wid * _RPW
    for c in range(_RPW // _CH):
        off = base + c * _CH
        pltpu.sync_copy(x_it_hbm.at[pl.ds(off, _CH)], idx_v)
        pltpu.async_copy(adj_hbm.at[idx_v], rows_v, sem).wait()
        pltpu.sync_copy(rows_v, out_hbm.at[pl.ds(off, _CH)])


def _gather_mask(x_it, adj_packed):
    mesh = plsc.VectorSubcoreMesh(core_axis_name="c", subcore_axis_name="s")
    return pl.kernel(
        _sc_gather,
        mesh=mesh,
        out_type=jax.ShapeDtypeStruct((N, VW), jnp.int32),
        scratch_types=[
            pltpu.VMEM((_CH,), jnp.int32),
            pltpu.VMEM((_CH, VW), jnp.int32),
            pltpu.SemaphoreType.DMA,
        ],
    )(x_it, adj_packed)


def kernel(a_ikt, product, Z_j, kappa, G_ijt, rho_jt, noise_v, x_it, adj, gamma_v):
    adj_u8 = jnp.pad(adj, ((0, 0), (0, VP - V))).astype(jnp.uint8)
    adj_packed = lax.bitcast_convert_type(
        adj_u8.reshape(V, VW, 4), jnp.int32)
    mask_w = _gather_mask(x_it, adj_packed)
    mask8 = lax.bitcast_convert_type(mask_w, jnp.int8).reshape(N, VP)

    grid = (N // BN,)
    row_spec = pl.BlockSpec((BN, V), lambda i: (i, 0))
    p, A, xn = pl.pallas_call(
        _tc_body,
        grid=grid,
        in_specs=[
            pl.BlockSpec((BN, K), lambda i: (i, 0)),
            pl.BlockSpec((V, K), lambda i: (0, 0)),
            pl.BlockSpec((1, V), lambda i: (0, 0)),
            pl.BlockSpec((BN, 1), lambda i: (i, 0)),
            row_spec,
            row_spec,
            row_spec,
            pl.BlockSpec((BN, VP), lambda i: (i, 0)),
            pl.BlockSpec((1, 1), lambda i: (0, 0)),
        ],
        out_specs=[
            row_spec,
            row_spec,
            pl.BlockSpec((BN, 1), lambda i: (i, 0)),
        ],
        out_shape=[
            jax.ShapeDtypeStruct((N, V), jnp.float32),
            jax.ShapeDtypeStruct((N, V), jnp.float32),
            jax.ShapeDtypeStruct((N, 1), jnp.int32),
        ],
    )(a_ikt, product, Z_j.reshape(1, V), kappa.reshape(N, 1),
      G_ijt, rho_jt, noise_v, mask8, gamma_v.reshape(1, 1))
    return p, A, xn.reshape(N)


# speedup vs baseline: 1.2596x; 1.2596x over previous
import functools

import jax
import jax.numpy as jnp
from jax import lax
from jax.experimental import pallas as pl
from jax.experimental.pallas import tpu as pltpu
from jax.experimental.pallas import tpu_sc as plsc

N, K, V = 16384, 64, 1000
BN = 512

_TINY = 1.1754943508222875e-38


def _gumbel_w_block(i):
    rowoff = (jax.lax.broadcasted_iota(jnp.uint32, (BN, 1), 0)
              + jnp.uint32(i * BN)) * jnp.uint32(V)
    col = jax.lax.broadcasted_iota(jnp.uint32, (BN, V), 1)
    c2 = rowoff + col
    k1 = jnp.uint32(0)
    k2 = jnp.uint32(42)
    ks0, ks1, ks2 = k1, k2, k1 ^ k2 ^ jnp.uint32(0x1BD11BDA)
    x0 = jnp.full((BN, V), ks0, jnp.uint32)
    x1 = c2 + ks1

    def rnd(x0, x1, r):
        x0 = x0 + x1
        x1 = (x1 << jnp.uint32(r)) | (x1 >> jnp.uint32(32 - r))
        return x0, x0 ^ x1

    rot_a = (13, 15, 26, 6)
    rot_b = (17, 29, 16, 24)
    for r in rot_a:
        x0, x1 = rnd(x0, x1, r)
    x0, x1 = x0 + ks1, x1 + ks2 + jnp.uint32(1)
    for r in rot_b:
        x0, x1 = rnd(x0, x1, r)
    x0, x1 = x0 + ks2, x1 + ks0 + jnp.uint32(2)
    for r in rot_a:
        x0, x1 = rnd(x0, x1, r)
    x0, x1 = x0 + ks0, x1 + ks1 + jnp.uint32(3)
    for r in rot_b:
        x0, x1 = rnd(x0, x1, r)
    x0, x1 = x0 + ks1, x1 + ks2 + jnp.uint32(4)
    for r in rot_a:
        x0, x1 = rnd(x0, x1, r)
    x0, x1 = x0 + ks2, x1 + ks0 + jnp.uint32(5)

    bits = x0 ^ x1
    fb = (bits >> jnp.uint32(9)) | jnp.uint32(0x3F800000)
    u01 = jax.lax.bitcast_convert_type(fb, jnp.float32) - jnp.float32(1.0)
    u = u01 + jnp.float32(_TINY)
    return -jnp.log(u)


def _tc_body(a_ref, prod_ref, z_ref, kap_ref, g_ref, rho_ref, noi_ref,
             m_ref, e4_ref, gamma_ref, p_ref, A_ref, xn_ref):
    ea = jnp.exp(a_ref[...])
    mm = lax.dot_general(ea, prod_ref[...], (((1,), (1,)), ((), ())),
                         preferred_element_type=jnp.float32)
    A_ref[...] = jnp.log(mm + 1.0)

    u = z_ref[...] + kap_ref[...] * g_ref[...] + gamma_ref[0, 0] * rho_ref[...] + noi_ref[...]

    m32 = m_ref[...]
    planes = [(((m32 >> (8 * k)) & 0xFF)).astype(jnp.bfloat16) for k in range(4)]
    bcat = jnp.concatenate(planes, axis=1)
    mexp = lax.dot_general(bcat, e4_ref[...], (((1,), (0,)), ((), ())),
                           preferred_element_type=jnp.float32)
    m = (mexp != 0)[:, :V]
    xmask = jnp.where(m, u, -jnp.inf)
    xmax = jnp.max(xmask, axis=1, keepdims=True)
    e = jnp.exp(xmask - xmax)
    s = jnp.sum(e, axis=1, keepdims=True)
    p = e / s
    p_ref[...] = p

    w = _gumbel_w_block(pl.program_id(0))
    r = jnp.where(m, (p + 1e-20) / w, 0.0)
    rmax = jnp.max(r, axis=1, keepdims=True)
    ii = lax.broadcasted_iota(jnp.int32, (BN, V), 1)
    idx = jnp.min(jnp.where(r == rmax, ii, jnp.int32(2**30)), axis=1)
    xn_ref[...] = idx[:, None]


VP = 1024
VW = VP // 4
_NW = 32
_RPW = N // _NW
_CH = 128


def _sc_gather(x_it_hbm, adj_hbm, out_hbm, idx_v, rows_v, sem):
    wid = lax.axis_index("s") * 2 + lax.axis_index("c")
    base = wid * _RPW
    for c in range(_RPW // _CH):
        off = base + c * _CH
        pltpu.sync_copy(x_it_hbm.at[pl.ds(off, _CH)], idx_v)
        pltpu.async_copy(adj_hbm.at[idx_v], rows_v, sem).wait()
        pltpu.sync_copy(rows_v, out_hbm.at[pl.ds(off, _CH)])


def _gather_mask(x_it, adj_packed):
    mesh = plsc.VectorSubcoreMesh(core_axis_name="c", subcore_axis_name="s")
    return pl.kernel(
        _sc_gather,
        mesh=mesh,
        out_type=jax.ShapeDtypeStruct((N, VW), jnp.int32),
        scratch_types=[
            pltpu.VMEM((_CH,), jnp.int32),
            pltpu.VMEM((_CH, VW), jnp.int32),
            pltpu.SemaphoreType.DMA,
        ],
    )(x_it, adj_packed)


def kernel(a_ikt, product, Z_j, kappa, G_ijt, rho_jt, noise_v, x_it, adj, gamma_v):
    adj_u8 = jnp.pad(adj, ((0, 0), (0, VP - V))).astype(jnp.uint8)
    adj_packed = lax.bitcast_convert_type(
        adj_u8.reshape(V, VW, 4), jnp.int32)
    mask_w = _gather_mask(x_it, adj_packed)
    rr = jnp.arange(VP, dtype=jnp.int32)[:, None]
    jj = jnp.arange(VP, dtype=jnp.int32)[None, :]
    e4 = (((jj >> 2) == (rr & (VW - 1))) & ((jj & 3) == (rr >> 8))
          ).astype(jnp.bfloat16)

    grid = (N // BN,)
    row_spec = pl.BlockSpec((BN, V), lambda i: (i, 0))
    p, A, xn = pl.pallas_call(
        _tc_body,
        grid=grid,
        in_specs=[
            pl.BlockSpec((BN, K), lambda i: (i, 0)),
            pl.BlockSpec((V, K), lambda i: (0, 0)),
            pl.BlockSpec((1, V), lambda i: (0, 0)),
            pl.BlockSpec((BN, 1), lambda i: (i, 0)),
            row_spec,
            row_spec,
            row_spec,
            pl.BlockSpec((BN, VW), lambda i: (i, 0)),
            pl.BlockSpec((VP, VP), lambda i: (0, 0)),
            pl.BlockSpec((1, 1), lambda i: (0, 0)),
        ],
        out_specs=[
            row_spec,
            row_spec,
            pl.BlockSpec((BN, 1), lambda i: (i, 0)),
        ],
        out_shape=[
            jax.ShapeDtypeStruct((N, V), jnp.float32),
            jax.ShapeDtypeStruct((N, V), jnp.float32),
            jax.ShapeDtypeStruct((N, 1), jnp.int32),
        ],
    )(a_ikt, product, Z_j.reshape(1, V), kappa.reshape(N, 1),
      G_ijt, rho_jt, noise_v, mask_w, e4, gamma_v.reshape(1, 1))
    return p, A, xn.reshape(N)
